# Initial kernel scaffold; baseline (speedup 1.0000x reference)
import jax
import jax.numpy as jnp
from jax.experimental import pallas as pl


def kernel(table, input_ids):
    raise NotImplementedError("write your pallas kernel here")


# speedup vs baseline: 1.8610x; 1.8610x over previous
import functools

import jax
import jax.numpy as jnp
from jax import lax
from jax.experimental import pallas as pl
from jax.experimental.pallas import tpu as pltpu
from jax.experimental.pallas import tpu_sc as plsc

_DIM = 768
_BATCH = 256
_SEQ = 512
_B = _BATCH * _SEQ
_NC = 2
_NS = 16
_NW = _NC * _NS
_BPW = _B // _NW
_CHUNK = 64
_NCHUNK = _BPW // _CHUNK
_NGROUP = _NCHUNK // 2

_mesh = plsc.VectorSubcoreMesh(core_axis_name="c", subcore_axis_name="s")


@functools.partial(
    pl.kernel,
    mesh=_mesh,
    out_type=jax.ShapeDtypeStruct((_B, _DIM), jnp.float32),
    scratch_types=[
        pltpu.VMEM((_NCHUNK, _CHUNK), jnp.int32),
        pltpu.VMEM((_CHUNK, _DIM), jnp.float32),
        pltpu.VMEM((_CHUNK, _DIM), jnp.float32),
        pltpu.SemaphoreType.DMA,
        pltpu.SemaphoreType.DMA,
        pltpu.SemaphoreType.DMA,
        pltpu.SemaphoreType.DMA,
    ],
)
def _emb_lookup(ids_hbm, table_hbm, out_hbm, idx_v, rows0, rows1,
                gsem0, gsem1, wsem0, wsem1):
    wid = lax.axis_index("s") * _NC + lax.axis_index("c")
    base = wid * _BPW

    pltpu.sync_copy(ids_hbm.at[wid], idx_v)

    pltpu.async_copy(table_hbm.at[idx_v.at[0]], rows0, gsem0)

    def group(g, _):
        c0 = 2 * g
        c1 = c0 + 1

        pltpu.make_async_copy(table_hbm.at[idx_v.at[c0]], rows0, gsem0).wait()
        pltpu.async_copy(
            rows0, out_hbm.at[pl.ds(base + c0 * _CHUNK, _CHUNK)], wsem0)

        @pl.when(g > 0)
        def _():
            pltpu.make_async_copy(
                rows1, out_hbm.at[pl.ds(base, _CHUNK)], wsem1).wait()

        pltpu.async_copy(table_hbm.at[idx_v.at[c1]], rows1, gsem1)

        pltpu.make_async_copy(table_hbm.at[idx_v.at[c1]], rows1, gsem1).wait()
        pltpu.async_copy(
            rows1, out_hbm.at[pl.ds(base + c1 * _CHUNK, _CHUNK)], wsem1)

        @pl.when(g + 1 < _NGROUP)
        def _():
            pltpu.make_async_copy(
                rows0, out_hbm.at[pl.ds(base, _CHUNK)], wsem0).wait()
            pltpu.async_copy(
                table_hbm.at[idx_v.at[2 * g + 2]], rows0, gsem0)

        return ()

    lax.fori_loop(0, _NGROUP, group, ())

    pltpu.make_async_copy(rows0, out_hbm.at[pl.ds(base, _CHUNK)], wsem0).wait()
    pltpu.make_async_copy(rows1, out_hbm.at[pl.ds(base, _CHUNK)], wsem1).wait()


def kernel(table, input_ids):
    ids = input_ids.reshape(_NW, _NCHUNK, _CHUNK).astype(jnp.int32)
    out = _emb_lookup(ids, table)
    return out.reshape(_BATCH, _SEQ, _DIM)
